# Initial kernel scaffold; baseline (speedup 1.0000x reference)
import jax
import jax.numpy as jnp
from jax.experimental import pallas as pl


def kernel(input_ids, attention_mask, d_embs, tok_embs, tok_embs_avg_weights, embs_avg_weights):
    raise NotImplementedError("write your pallas kernel here")


# speedup vs baseline: 1.8409x; 1.8409x over previous
import jax
import jax.numpy as jnp
from jax import lax
from jax.experimental import pallas as pl
from jax.experimental.pallas import tpu as pltpu
from jax.experimental.pallas import tpu_sc as plsc


B, L, V, D, NDOCS = 4096, 32, 30522, 768, 10
NEMBS = NDOCS + 1
NW = 32
RPW = B // NW
VPAD = 30528
DC = D // 16


def _sc_body(ids_hbm, dembs_hbm, table_hbm, tokw_hbm, ew_hbm, out_hbm,
             tokw_v, ids_v, ew_v,
             rows_v0, rows_v1, dbuf_v0, dbuf_v1, acc_v0, acc_v1,
             rsem0, rsem1, dsem0, dsem1, osem0, osem1):
    cid = lax.axis_index("c")
    sid = lax.axis_index("s")
    wid = sid * 2 + cid
    base = wid * RPW

    pltpu.sync_copy(tokw_hbm, tokw_v)
    pltpu.sync_copy(ids_hbm.at[pl.ds(base * L, RPW * L)], ids_v)
    pltpu.sync_copy(ew_hbm, ew_v)

    ew = ew_v[...]
    ewe = jnp.exp(ew - jnp.full((16,), jnp.max(ew)))
    ws = ewe / jnp.full((16,), jnp.sum(ewe))
    ew0 = ws[0]
    dws = [ws[1 + k] for k in range(NDOCS)]

    def start_row(r, rowsbuf, dbuf, rsem, dsem):
        off = pl.multiple_of(r * L, L)
        pltpu.make_async_copy(
            table_hbm.at[ids_v.at[pl.ds(off, L)]], rowsbuf, rsem).start()
        pltpu.make_async_copy(
            dembs_hbm.at[pl.ds((base + r) * NDOCS * D, NDOCS * D)],
            dbuf, dsem).start()

    def wait_row(r, rowsbuf, dbuf, rsem, dsem):
        off = pl.multiple_of(r * L, L)
        pltpu.make_async_copy(
            table_hbm.at[ids_v.at[pl.ds(off, L)]], rowsbuf, rsem).wait()
        pltpu.make_async_copy(
            dembs_hbm.at[pl.ds((base + r) * NDOCS * D, NDOCS * D)],
            dbuf, dsem).wait()

    def compute_row(r, rowsbuf, dbuf, accbuf, osem, not_first):
        off = pl.multiple_of(r * L, L)
        i1 = ids_v[pl.ds(off, 16)]
        i2 = ids_v[pl.ds(off + 16, 16)]
        w1 = plsc.load_gather(tokw_v, [i1])
        w2 = plsc.load_gather(tokw_v, [i2])
        m = jnp.full((16,), jnp.max(jnp.maximum(w1, w2)))
        e1 = jnp.exp(w1 - m)
        e2 = jnp.exp(w2 - m)
        coef = jnp.full((16,), ew0) / jnp.full((16,), jnp.sum(e1 + e2))
        e1 = e1 * coef
        e2 = e2 * coef
        els = [e1[i] for i in range(16)] + [e2[i] for i in range(16)]

        @pl.when(not_first)
        def _():
            pltpu.make_async_copy(
                accbuf, out_hbm.at[pl.ds((base + r) * D, D)], osem).wait()

        def chunk_step(c, carry):
            cs = pl.multiple_of(c * 16, 16)
            acc = dws[0] * dbuf[pl.ds(cs, 16)]
            for k in range(1, NDOCS):
                acc = acc + dws[k] * dbuf[pl.ds(k * D + cs, 16)]
            for l in range(L):
                acc = acc + els[l] * rowsbuf[l, pl.ds(cs, 16)]
            accbuf[pl.ds(cs, 16)] = acc
            return carry
        lax.fori_loop(0, DC, chunk_step, 0)

        pltpu.make_async_copy(
            accbuf, out_hbm.at[pl.ds((base + r) * D, D)], osem).start()

    half = RPW // 2
    start_row(0, rows_v0, dbuf_v0, rsem0, dsem0)

    def outer(rr, carry):
        r0 = rr * 2
        start_row(r0 + 1, rows_v1, dbuf_v1, rsem1, dsem1)
        wait_row(r0, rows_v0, dbuf_v0, rsem0, dsem0)
        compute_row(r0, rows_v0, dbuf_v0, acc_v0, osem0, rr > 0)

        @pl.when(rr + 1 < half)
        def _():
            start_row(r0 + 2, rows_v0, dbuf_v0, rsem0, dsem0)
        wait_row(r0 + 1, rows_v1, dbuf_v1, rsem1, dsem1)
        compute_row(r0 + 1, rows_v1, dbuf_v1, acc_v1, osem1, rr > 0)
        return carry
    lax.fori_loop(0, half, outer, 0)

    pltpu.make_async_copy(
        acc_v0, out_hbm.at[pl.ds((base + RPW - 2) * D, D)], osem0).wait()
    pltpu.make_async_copy(
        acc_v1, out_hbm.at[pl.ds((base + RPW - 1) * D, D)], osem1).wait()


@jax.jit
def _run(ids_flat, dembs_flat, tok_embs, tokw_pad, ew_pad):
    mesh = plsc.VectorSubcoreMesh(core_axis_name="c", subcore_axis_name="s")
    f = pl.kernel(
        _sc_body,
        out_type=jax.ShapeDtypeStruct((B * D,), jnp.float32),
        mesh=mesh,
        compiler_params=pltpu.CompilerParams(needs_layout_passes=False),
        scratch_types=[
            pltpu.VMEM((VPAD,), jnp.float32),
            pltpu.VMEM((RPW * L,), jnp.int32),
            pltpu.VMEM((16,), jnp.float32),
            pltpu.VMEM((L, D), jnp.float32),
            pltpu.VMEM((L, D), jnp.float32),
            pltpu.VMEM((NDOCS * D,), jnp.float32),
            pltpu.VMEM((NDOCS * D,), jnp.float32),
            pltpu.VMEM((D,), jnp.float32),
            pltpu.VMEM((D,), jnp.float32),
            pltpu.SemaphoreType.DMA,
            pltpu.SemaphoreType.DMA,
            pltpu.SemaphoreType.DMA,
            pltpu.SemaphoreType.DMA,
            pltpu.SemaphoreType.DMA,
            pltpu.SemaphoreType.DMA,
        ],
    )
    return f(ids_flat, dembs_flat, tok_embs, tokw_pad, ew_pad)


def kernel(input_ids, attention_mask, d_embs, tok_embs, tok_embs_avg_weights,
           embs_avg_weights):
    del attention_mask
    ids_flat = input_ids.reshape(B * L)
    dembs_flat = d_embs.reshape(B * NDOCS * D)
    tokw_pad = jnp.pad(tok_embs_avg_weights, (0, VPAD - V))
    ew_pad = jnp.pad(embs_avg_weights, (0, 16 - NEMBS), constant_values=-1e30)
    out = _run(ids_flat, dembs_flat, tok_embs, tokw_pad, ew_pad)
    return out.reshape(B, D)


# speedup vs baseline: 2.1218x; 1.1526x over previous
import jax
import jax.numpy as jnp
from jax import lax
from jax.experimental import pallas as pl
from jax.experimental.pallas import tpu as pltpu
from jax.experimental.pallas import tpu_sc as plsc


B, L, V, D, NDOCS = 4096, 32, 30522, 768, 10
NEMBS = NDOCS + 1
NW = 32
RPW = B // NW
VPAD = 30528
DC = D // 16
BLK = 256


def _doc_body(ew_ref, d_ref, o_ref):
    ew = ew_ref[...]
    e = jnp.exp(ew - jnp.max(ew))
    ws = e / jnp.sum(e)
    acc = ws[0:1, 1:2] * d_ref[:, 0, :]
    for k in range(1, NDOCS):
        acc = acc + ws[0:1, k + 1:k + 2] * d_ref[:, k, :]
    o_ref[...] = acc


def _sc_body(ids_hbm, docpart_hbm, table_hbm, tokw_hbm, ew_hbm, out_hbm,
             tokw_v, ids_v, ew_v,
             rows_v0, rows_v1, dbuf_v0, dbuf_v1, acc_v0, acc_v1,
             rsem0, rsem1, dsem0, dsem1, osem0, osem1):
    cid = lax.axis_index("c")
    sid = lax.axis_index("s")
    wid = sid * 2 + cid
    base = wid * RPW

    pltpu.sync_copy(tokw_hbm, tokw_v)
    pltpu.sync_copy(ids_hbm.at[pl.ds(base * L, RPW * L)], ids_v)
    pltpu.sync_copy(ew_hbm, ew_v)

    ew = ew_v[...]
    ewe = jnp.exp(ew - jnp.full((16,), jnp.max(ew)))
    ws = ewe / jnp.full((16,), jnp.sum(ewe))
    ew0 = ws[0]

    def start_row(r, rowsbuf, dbuf, rsem, dsem):
        off = pl.multiple_of(r * L, L)
        pltpu.make_async_copy(
            table_hbm.at[ids_v.at[pl.ds(off, L)]], rowsbuf, rsem).start()
        pltpu.make_async_copy(
            docpart_hbm.at[pl.ds((base + r) * D, D)], dbuf, dsem).start()

    def wait_row(r, rowsbuf, dbuf, rsem, dsem):
        off = pl.multiple_of(r * L, L)
        pltpu.make_async_copy(
            table_hbm.at[ids_v.at[pl.ds(off, L)]], rowsbuf, rsem).wait()
        pltpu.make_async_copy(
            docpart_hbm.at[pl.ds((base + r) * D, D)], dbuf, dsem).wait()

    def compute_row(r, rowsbuf, dbuf, accbuf, osem, not_first):
        off = pl.multiple_of(r * L, L)
        i1 = ids_v[pl.ds(off, 16)]
        i2 = ids_v[pl.ds(off + 16, 16)]
        w1 = plsc.load_gather(tokw_v, [i1])
        w2 = plsc.load_gather(tokw_v, [i2])
        m = jnp.full((16,), jnp.max(jnp.maximum(w1, w2)))
        e1 = jnp.exp(w1 - m)
        e2 = jnp.exp(w2 - m)
        coef = jnp.full((16,), ew0) / jnp.full((16,), jnp.sum(e1 + e2))
        e1 = e1 * coef
        e2 = e2 * coef
        els = [e1[i] for i in range(16)] + [e2[i] for i in range(16)]

        @pl.when(not_first)
        def _():
            pltpu.make_async_copy(
                accbuf, out_hbm.at[pl.ds((base + r) * D, D)], osem).wait()

        def chunk_step(c, carry):
            cs = pl.multiple_of(c * 16, 16)
            acc = dbuf[pl.ds(cs, 16)]
            for l in range(L):
                acc = acc + els[l] * rowsbuf[l, pl.ds(cs, 16)]
            accbuf[pl.ds(cs, 16)] = acc
            return carry
        lax.fori_loop(0, DC, chunk_step, 0)

        pltpu.make_async_copy(
            accbuf, out_hbm.at[pl.ds((base + r) * D, D)], osem).start()

    half = RPW // 2
    start_row(0, rows_v0, dbuf_v0, rsem0, dsem0)

    def outer(rr, carry):
        r0 = rr * 2
        start_row(r0 + 1, rows_v1, dbuf_v1, rsem1, dsem1)
        wait_row(r0, rows_v0, dbuf_v0, rsem0, dsem0)
        compute_row(r0, rows_v0, dbuf_v0, acc_v0, osem0, rr > 0)

        @pl.when(rr + 1 < half)
        def _():
            start_row(r0 + 2, rows_v0, dbuf_v0, rsem0, dsem0)
        wait_row(r0 + 1, rows_v1, dbuf_v1, rsem1, dsem1)
        compute_row(r0 + 1, rows_v1, dbuf_v1, acc_v1, osem1, rr > 0)
        return carry
    lax.fori_loop(0, half, outer, 0)

    pltpu.make_async_copy(
        acc_v0, out_hbm.at[pl.ds((base + RPW - 2) * D, D)], osem0).wait()
    pltpu.make_async_copy(
        acc_v1, out_hbm.at[pl.ds((base + RPW - 1) * D, D)], osem1).wait()


@jax.jit
def _run(ids_flat, d_embs, tok_embs, tokw_pad, ew_pad):
    docpart = pl.pallas_call(
        _doc_body,
        grid=(B // BLK,),
        in_specs=[
            pl.BlockSpec((1, 16), lambda i: (0, 0)),
            pl.BlockSpec((BLK, NDOCS, D), lambda i: (i, 0, 0)),
        ],
        out_specs=pl.BlockSpec((BLK, D), lambda i: (i, 0)),
        out_shape=jax.ShapeDtypeStruct((B, D), jnp.float32),
    )(ew_pad.reshape(1, 16), d_embs)

    mesh = plsc.VectorSubcoreMesh(core_axis_name="c", subcore_axis_name="s")
    f = pl.kernel(
        _sc_body,
        out_type=jax.ShapeDtypeStruct((B * D,), jnp.float32),
        mesh=mesh,
        compiler_params=pltpu.CompilerParams(needs_layout_passes=False),
        scratch_types=[
            pltpu.VMEM((VPAD,), jnp.float32),
            pltpu.VMEM((RPW * L,), jnp.int32),
            pltpu.VMEM((16,), jnp.float32),
            pltpu.VMEM((L, D), jnp.float32),
            pltpu.VMEM((L, D), jnp.float32),
            pltpu.VMEM((D,), jnp.float32),
            pltpu.VMEM((D,), jnp.float32),
            pltpu.VMEM((D,), jnp.float32),
            pltpu.VMEM((D,), jnp.float32),
            pltpu.SemaphoreType.DMA,
            pltpu.SemaphoreType.DMA,
            pltpu.SemaphoreType.DMA,
            pltpu.SemaphoreType.DMA,
            pltpu.SemaphoreType.DMA,
            pltpu.SemaphoreType.DMA,
        ],
    )
    out = f(ids_flat, docpart.reshape(B * D), tok_embs, tokw_pad, ew_pad)
    return out.reshape(B, D)


def kernel(input_ids, attention_mask, d_embs, tok_embs, tok_embs_avg_weights,
           embs_avg_weights):
    del attention_mask
    ids_flat = input_ids.reshape(B * L)
    tokw_pad = jnp.pad(tok_embs_avg_weights, (0, VPAD - V))
    ew_pad = jnp.pad(embs_avg_weights, (0, 16 - NEMBS), constant_values=-1e30)
    return _run(ids_flat, d_embs, tok_embs, tokw_pad, ew_pad)
